# baseline (speedup 1.0000x reference)
import jax
import jax.numpy as jnp
from jax.experimental import pallas as pl

N = 512
M = N * 27 * 3
R = M // 128
BI = 16


def _body(pj0, pj1, pj2, sv0, sv1, sv2, e0w, e1w, rjw,
          pxr, pyr, pzr, rir, out_ref):
    e0 = e0w[...][None, :, :] != 0
    e1 = e1w[...][None, :, :] != 0
    px = pxr[...][:, None, :]
    py = pyr[...][:, None, :]
    pz = pzr[...][:, None, :]
    d0 = (pj0[...][None, :, :] - px) + sv0[...][None, :, :]
    d1 = (pj1[...][None, :, :] - py) + sv1[...][None, :, :]
    d2 = (pj2[...][None, :, :] - pz) + sv2[...][None, :, :]
    rs = jnp.sqrt((d0 * d0 + d1 * d1) + d2 * d2)
    rj = rjw[...]
    gcut = 2.0 * jnp.max(rj)
    cut = jnp.minimum(rir[...][:, None, :] + rj[None, :, :], gcut)
    mask = (rs <= cut) & (rs > 1e-8)
    disp = jnp.where(e0, d0, jnp.where(e1, d1, d2))
    out_ref[...] = jnp.where(mask, disp, 0.0)


def _field(PJ0, PJ1, PJ2, SV0, SV1, SV2, E0, E1, RJW,
           PX, PY, PZ, RI):
    wrap = pl.BlockSpec((R, 128), lambda i: (0, 0))
    rows = pl.BlockSpec((BI, 128), lambda i: (i, 0))
    return pl.pallas_call(
        _body,
        grid=(N // BI,),
        in_specs=[wrap] * 9 + [rows] * 4,
        out_specs=pl.BlockSpec((BI, R, 128), lambda i: (i, 0, 0)),
        out_shape=jax.ShapeDtypeStruct((N, R, 128), jnp.float32),
    )(PJ0, PJ1, PJ2, SV0, SV1, SV2, E0, E1, RJW, PX, PY, PZ, RI)


def kernel(positions, cell, radii_table, numbers):
    positions = positions.astype(jnp.float32)
    s = jnp.arange(-1, 2, dtype=positions.dtype)
    g = jnp.meshgrid(s, s, s, indexing="ij")
    shifts = jnp.stack(g, axis=-1).reshape(-1, 3)
    shift_vecs = shifts @ cell

    radii = jnp.take(radii_table, numbers, axis=0).astype(jnp.float32)

    m = jnp.arange(M, dtype=jnp.int32)
    jj = m // 81
    kk = m % 81
    ss = kk // 3
    cc = kk % 3

    PJ0 = jnp.take(positions[:, 0], jj).reshape(R, 128)
    PJ1 = jnp.take(positions[:, 1], jj).reshape(R, 128)
    PJ2 = jnp.take(positions[:, 2], jj).reshape(R, 128)
    SV0 = jnp.take(shift_vecs[:, 0], ss).reshape(R, 128)
    SV1 = jnp.take(shift_vecs[:, 1], ss).reshape(R, 128)
    SV2 = jnp.take(shift_vecs[:, 2], ss).reshape(R, 128)
    E0 = (cc == 0).astype(jnp.int32).reshape(R, 128)
    E1 = (cc == 1).astype(jnp.int32).reshape(R, 128)
    RJW = jnp.take(radii, jj).reshape(R, 128)

    PX = jnp.broadcast_to(positions[:, 0:1], (N, 128))
    PY = jnp.broadcast_to(positions[:, 1:2], (N, 128))
    PZ = jnp.broadcast_to(positions[:, 2:3], (N, 128))
    RI = jnp.broadcast_to(radii[:, None], (N, 128))

    out = _field(PJ0, PJ1, PJ2, SV0, SV1, SV2, E0, E1, RJW,
                 PX, PY, PZ, RI)
    return out.reshape(N, N, 27, 3)


# speedup vs baseline: 3.3875x; 3.3875x over previous
import jax
import jax.numpy as jnp
from jax.experimental import pallas as pl

N = 512
M = N * 27 * 3
R = M // 128
BI = 16


def _body(pj0, pj1, pj2, sv0, sv1, sv2, e0w, e1w, rjw,
          pxr, pyr, pzr, rir, out_ref):
    e0 = e0w[...][None, :, :] != 0
    e1 = e1w[...][None, :, :] != 0
    px = pxr[...][:, None, :]
    py = pyr[...][:, None, :]
    pz = pzr[...][:, None, :]
    d0 = (pj0[...][None, :, :] - px) + sv0[...][None, :, :]
    d1 = (pj1[...][None, :, :] - py) + sv1[...][None, :, :]
    d2 = (pj2[...][None, :, :] - pz) + sv2[...][None, :, :]
    rs = jnp.sqrt((d0 * d0 + d1 * d1) + d2 * d2)
    rj = rjw[...]
    gcut = 2.0 * jnp.max(rj)
    cut = jnp.minimum(rir[...][:, None, :] + rj[None, :, :], gcut)
    mask = (rs <= cut) & (rs > 1e-8)
    disp = jnp.where(e0, d0, jnp.where(e1, d1, d2))
    out_ref[...] = jnp.where(mask, disp, 0.0)


def _field(PJ0, PJ1, PJ2, SV0, SV1, SV2, E0, E1, RJW,
           PX, PY, PZ, RI):
    wrap = pl.BlockSpec((R, 128), lambda i: (0, 0))
    rows = pl.BlockSpec((BI, 128), lambda i: (i, 0))
    return pl.pallas_call(
        _body,
        grid=(N // BI,),
        in_specs=[wrap] * 9 + [rows] * 4,
        out_specs=pl.BlockSpec((BI, R, 128), lambda i: (i, 0, 0)),
        out_shape=jax.ShapeDtypeStruct((N, R, 128), jnp.float32),
    )(PJ0, PJ1, PJ2, SV0, SV1, SV2, E0, E1, RJW, PX, PY, PZ, RI)


def kernel(positions, cell, radii_table, numbers):
    positions = positions.astype(jnp.float32)
    s = jnp.arange(-1, 2, dtype=positions.dtype)
    g = jnp.meshgrid(s, s, s, indexing="ij")
    shifts = jnp.stack(g, axis=-1).reshape(-1, 3)
    shift_vecs = shifts @ cell

    radii = jnp.take(radii_table, numbers, axis=0).astype(jnp.float32)

    cc = jnp.arange(M, dtype=jnp.int32) % 3

    def rep_j(v):
        return jnp.broadcast_to(v[:, None], (N, 81)).reshape(R, 128)

    def rep_s(v):
        row = jnp.repeat(v, 3).reshape(1, 81)
        return jnp.broadcast_to(row, (N, 81)).reshape(R, 128)

    PJ0 = rep_j(positions[:, 0])
    PJ1 = rep_j(positions[:, 1])
    PJ2 = rep_j(positions[:, 2])
    SV0 = rep_s(shift_vecs[:, 0])
    SV1 = rep_s(shift_vecs[:, 1])
    SV2 = rep_s(shift_vecs[:, 2])
    E0 = (cc == 0).astype(jnp.int32).reshape(R, 128)
    E1 = (cc == 1).astype(jnp.int32).reshape(R, 128)
    RJW = rep_j(radii)

    PX = jnp.broadcast_to(positions[:, 0:1], (N, 128))
    PY = jnp.broadcast_to(positions[:, 1:2], (N, 128))
    PZ = jnp.broadcast_to(positions[:, 2:3], (N, 128))
    RI = jnp.broadcast_to(radii[:, None], (N, 128))

    out = _field(PJ0, PJ1, PJ2, SV0, SV1, SV2, E0, E1, RJW,
                 PX, PY, PZ, RI)
    return out.reshape(N, N, 27, 3)


# speedup vs baseline: 22.1612x; 6.5420x over previous
import jax
import jax.numpy as jnp
from jax.experimental import pallas as pl

N = 512
BI = 8


def _body(pj_ref, sv_ref, rj_ref, pi_ref, ri_ref, out_ref):
    pj = pj_ref[...]
    pj0 = pj[0:1, :]
    pj1 = pj[1:2, :]
    pj2 = pj[2:3, :]
    pi = pi_ref[...]
    pi0 = pi[:, 0:1]
    pi1 = pi[:, 1:2]
    pi2 = pi[:, 2:3]
    dx0 = pj0 - pi0
    dx1 = pj1 - pi1
    dx2 = pj2 - pi2
    rj = rj_ref[...][0:1, :]
    gcut = 2.0 * jnp.max(rj)
    cut = jnp.minimum(ri_ref[...][:, 0:1] + rj, gcut)
    eps = jnp.float32(1e-8)
    for s in range(27):
        d0 = dx0 + sv_ref[s, 0]
        d1 = dx1 + sv_ref[s, 1]
        d2 = dx2 + sv_ref[s, 2]
        rs = jnp.sqrt((d0 * d0 + d1 * d1) + d2 * d2)
        mask = (rs <= cut) & (rs > eps)
        out_ref[s, 0, :, :] = jnp.where(mask, d0, 0.0)
        out_ref[s, 1, :, :] = jnp.where(mask, d1, 0.0)
        out_ref[s, 2, :, :] = jnp.where(mask, d2, 0.0)


def _field(PJ, SV, RJ, PI, RI):
    return pl.pallas_call(
        _body,
        grid=(N // BI,),
        in_specs=[
            pl.BlockSpec((8, N), lambda i: (0, 0)),
            pl.BlockSpec((32, 128), lambda i: (0, 0)),
            pl.BlockSpec((8, N), lambda i: (0, 0)),
            pl.BlockSpec((BI, 128), lambda i: (i, 0)),
            pl.BlockSpec((BI, 128), lambda i: (i, 0)),
        ],
        out_specs=pl.BlockSpec((27, 3, BI, N), lambda i: (0, 0, i, 0)),
        out_shape=jax.ShapeDtypeStruct((27, 3, N, N), jnp.float32),
    )(PJ, SV, RJ, PI, RI)


def kernel(positions, cell, radii_table, numbers):
    positions = positions.astype(jnp.float32)
    s = jnp.arange(-1, 2, dtype=positions.dtype)
    g = jnp.meshgrid(s, s, s, indexing="ij")
    shifts = jnp.stack(g, axis=-1).reshape(-1, 3)
    shift_vecs = shifts @ cell

    radii = jnp.take(radii_table, numbers, axis=0).astype(jnp.float32)

    PJ = jnp.pad(positions.T, ((0, 5), (0, 0)))
    SV = jnp.pad(shift_vecs, ((0, 5), (0, 125)))
    RJ = jnp.broadcast_to(radii[None, :], (8, N))
    PI = jnp.pad(positions, ((0, 0), (0, 125)))
    RI = jnp.broadcast_to(radii[:, None], (N, 128))

    out = _field(PJ, SV, RJ, PI, RI)
    return jnp.transpose(out, (2, 3, 0, 1))


# speedup vs baseline: 33.0929x; 1.4933x over previous
import jax
import jax.numpy as jnp
from jax.experimental import pallas as pl

N = 512
BI = 32


def _body(pj_ref, sv_ref, rj_ref, pi_ref, ri_ref, out_ref):
    pj = pj_ref[...]
    pj0 = pj[0:1, :]
    pj1 = pj[1:2, :]
    pj2 = pj[2:3, :]
    pi = pi_ref[...]
    pi0 = pi[:, 0:1]
    pi1 = pi[:, 1:2]
    pi2 = pi[:, 2:3]
    dx0 = pj0 - pi0
    dx1 = pj1 - pi1
    dx2 = pj2 - pi2
    rj = rj_ref[...][0:1, :]
    gcut = 2.0 * jnp.max(rj)
    cut = jnp.minimum(ri_ref[...][:, 0:1] + rj, gcut)
    eps = jnp.float32(1e-8)
    for s in range(27):
        d0 = dx0 + sv_ref[s, 0]
        d1 = dx1 + sv_ref[s, 1]
        d2 = dx2 + sv_ref[s, 2]
        rs = jnp.sqrt((d0 * d0 + d1 * d1) + d2 * d2)
        mask = (rs <= cut) & (rs > eps)
        out_ref[s, 0, :, :] = jnp.where(mask, d0, 0.0)
        out_ref[s, 1, :, :] = jnp.where(mask, d1, 0.0)
        out_ref[s, 2, :, :] = jnp.where(mask, d2, 0.0)


def _field(PJ, SV, RJ, PI, RI):
    return pl.pallas_call(
        _body,
        grid=(N // BI,),
        in_specs=[
            pl.BlockSpec((8, N), lambda i: (0, 0)),
            pl.BlockSpec((32, 128), lambda i: (0, 0)),
            pl.BlockSpec((8, N), lambda i: (0, 0)),
            pl.BlockSpec((BI, 128), lambda i: (i, 0)),
            pl.BlockSpec((BI, 128), lambda i: (i, 0)),
        ],
        out_specs=pl.BlockSpec((27, 3, BI, N), lambda i: (0, 0, i, 0)),
        out_shape=jax.ShapeDtypeStruct((27, 3, N, N), jnp.float32),
    )(PJ, SV, RJ, PI, RI)


def kernel(positions, cell, radii_table, numbers):
    positions = positions.astype(jnp.float32)
    s = jnp.arange(-1, 2, dtype=positions.dtype)
    g = jnp.meshgrid(s, s, s, indexing="ij")
    shifts = jnp.stack(g, axis=-1).reshape(-1, 3)
    shift_vecs = shifts @ cell

    radii = jnp.take(radii_table, numbers, axis=0).astype(jnp.float32)

    PJ = jnp.pad(positions.T, ((0, 5), (0, 0)))
    SV = jnp.pad(shift_vecs, ((0, 5), (0, 125)))
    RJ = jnp.broadcast_to(radii[None, :], (8, N))
    PI = jnp.pad(positions, ((0, 0), (0, 125)))
    RI = jnp.broadcast_to(radii[:, None], (N, 128))

    out = _field(PJ, SV, RJ, PI, RI)
    return jnp.transpose(out, (2, 3, 0, 1))


# speedup vs baseline: 33.7936x; 1.0212x over previous
import jax
import jax.numpy as jnp
from jax.experimental import pallas as pl

N = 512
BI = 64


def _body(pj_ref, sv_ref, rj_ref, pi_ref, ri_ref, out_ref):
    pj = pj_ref[...]
    pj0 = pj[0:1, :]
    pj1 = pj[1:2, :]
    pj2 = pj[2:3, :]
    pi = pi_ref[...]
    pi0 = pi[:, 0:1]
    pi1 = pi[:, 1:2]
    pi2 = pi[:, 2:3]
    dx0 = pj0 - pi0
    dx1 = pj1 - pi1
    dx2 = pj2 - pi2
    rj = rj_ref[...][0:1, :]
    gcut = 2.0 * jnp.max(rj)
    cut = jnp.minimum(ri_ref[...][:, 0:1] + rj, gcut)
    eps = jnp.float32(1e-8)
    for s in range(27):
        d0 = dx0 + sv_ref[s, 0]
        d1 = dx1 + sv_ref[s, 1]
        d2 = dx2 + sv_ref[s, 2]
        rs = jnp.sqrt((d0 * d0 + d1 * d1) + d2 * d2)
        mask = (rs <= cut) & (rs > eps)
        out_ref[s, 0, :, :] = jnp.where(mask, d0, 0.0)
        out_ref[s, 1, :, :] = jnp.where(mask, d1, 0.0)
        out_ref[s, 2, :, :] = jnp.where(mask, d2, 0.0)


def _field(PJ, SV, RJ, PI, RI):
    return pl.pallas_call(
        _body,
        grid=(N // BI,),
        in_specs=[
            pl.BlockSpec((8, N), lambda i: (0, 0)),
            pl.BlockSpec((32, 128), lambda i: (0, 0)),
            pl.BlockSpec((8, N), lambda i: (0, 0)),
            pl.BlockSpec((BI, 128), lambda i: (i, 0)),
            pl.BlockSpec((BI, 128), lambda i: (i, 0)),
        ],
        out_specs=pl.BlockSpec((27, 3, BI, N), lambda i: (0, 0, i, 0)),
        out_shape=jax.ShapeDtypeStruct((27, 3, N, N), jnp.float32),
    )(PJ, SV, RJ, PI, RI)


def kernel(positions, cell, radii_table, numbers):
    positions = positions.astype(jnp.float32)
    s = jnp.arange(-1, 2, dtype=positions.dtype)
    g = jnp.meshgrid(s, s, s, indexing="ij")
    shifts = jnp.stack(g, axis=-1).reshape(-1, 3)
    shift_vecs = shifts @ cell

    radii = jnp.take(radii_table, numbers, axis=0).astype(jnp.float32)

    PJ = jnp.pad(positions.T, ((0, 5), (0, 0)))
    SV = jnp.pad(shift_vecs, ((0, 5), (0, 125)))
    RJ = jnp.broadcast_to(radii[None, :], (8, N))
    PI = jnp.pad(positions, ((0, 0), (0, 125)))
    RI = jnp.broadcast_to(radii[:, None], (N, 128))

    out = _field(PJ, SV, RJ, PI, RI)
    return jnp.transpose(out, (2, 3, 0, 1))
